# baseline (speedup 1.0000x reference)
import jax
import jax.numpy as jnp
from jax import lax
from jax.experimental import pallas as pl
from jax.experimental.pallas import tpu as pltpu
from jax.experimental.pallas import tpu_sc as plsc

B = 16384
L = 200
D = 64
DP = 128
HID = 256
OUT = 2

NC = 2
NS = 16
NW = NC * NS
ROWS_PER_W = B // NW
LANES = 16
W = 32
NG = ROWS_PER_W // W
UNROLL = 8
NACC = 2 * (D // LANES)


def _pool_body(idx_hbm, table_hbm, out_hbm, ib, gb0, gb1, ab, sg0, sg1):
    wid = lax.axis_index("s") * NC + lax.axis_index("c")
    base = wid * ROWS_PER_W
    ii = jnp.arange(LANES, dtype=jnp.int32)
    himask = jnp.full((LANES,), -65536, dtype=jnp.int32)

    @pl.loop(0, NG)
    def _(g):
        rowbase = base + g * W
        pltpu.sync_copy(idx_hbm.at[pl.ds(rowbase, W)], ib)
        pltpu.async_copy(table_hbm.at[ib.at[0]], gb0, sg0)
        for w in range(W):
            gb, sg = (gb0, sg0) if w % 2 == 0 else (gb1, sg1)
            ngb, nsg = (gb1, sg1) if w % 2 == 0 else (gb0, sg0)
            if w + 1 < W:
                pltpu.async_copy(table_hbm.at[ib.at[w + 1]], ngb, nsg)
            pltpu.make_async_copy(table_hbm.at[ib.at[w]], gb, sg).wait()

            def body(i, acc, gb=gb):
                new = list(acc)
                for u in range(UNROLL):
                    l = i * UNROLL + u
                    for c in range(2):
                        w32 = plsc.bitcast(
                            gb[l, pl.ds(c * 2 * LANES, 2 * LANES)], jnp.int32)
                        lo = plsc.bitcast(w32 << 16, jnp.float32)
                        hi = plsc.bitcast(w32 & himask, jnp.float32)
                        k = (u % 2) * (NACC // 2) + 2 * c
                        new[k] = new[k] + lo
                        new[k + 1] = new[k + 1] + hi
                return tuple(new)

            zero = jnp.zeros((LANES,), jnp.float32)
            acc = lax.fori_loop(0, L // UNROLL, body, (zero,) * NACC)
            for c in range(2):
                lo = acc[2 * c] + acc[NACC // 2 + 2 * c]
                hi = acc[2 * c + 1] + acc[NACC // 2 + 2 * c + 1]
                plsc.store_scatter(ab.at[w], [c * 2 * LANES + 2 * ii], lo)
                plsc.store_scatter(ab.at[w], [c * 2 * LANES + 2 * ii + 1], hi)
        pltpu.sync_copy(ab, out_hbm.at[pl.ds(rowbase, W)])


def _pool(indices, table_bf):
    mesh = plsc.VectorSubcoreMesh(core_axis_name="c", subcore_axis_name="s")
    k = pl.kernel(
        _pool_body,
        out_type=jax.ShapeDtypeStruct((B, D), jnp.float32),
        mesh=mesh,
        compiler_params=pltpu.CompilerParams(
            use_tc_tiling_on_sc=False, needs_layout_passes=False),
        scratch_types=[
            pltpu.VMEM((W, L), jnp.int32),
            pltpu.VMEM((L, DP), jnp.bfloat16),
            pltpu.VMEM((L, DP), jnp.bfloat16),
            pltpu.VMEM((W, D), jnp.float32),
            pltpu.SemaphoreType.DMA,
            pltpu.SemaphoreType.DMA,
        ],
    )
    return k(indices, table_bf)


def _lengths_body(idx_ref, len_ref):
    idx = idx_ref[...]
    len_ref[...] = jnp.sum((idx != 0).astype(jnp.float32), axis=1,
                           keepdims=True)


def _lengths(indices):
    BLK = 2048
    return pl.pallas_call(
        _lengths_body,
        grid=(B // BLK,),
        in_specs=[pl.BlockSpec((BLK, L), lambda i: (i, 0))],
        out_specs=pl.BlockSpec((BLK, 1), lambda i: (i, 0)),
        out_shape=jax.ShapeDtypeStruct((B, 1), jnp.float32),
    )(indices)


def _tail_body(sums_ref, len_ref, row0_ref, gamma_ref, beta_ref,
               w1_ref, b1_ref, w2_ref, b2_ref, out_ref):
    lengths = len_ref[...]
    n_zeros = jnp.float32(L) - lengths
    s = sums_ref[...] - n_zeros * row0_ref[...]
    avg = s / jnp.maximum(lengths, 1.0)
    mu = jnp.mean(avg, axis=-1, keepdims=True)
    var = jnp.mean((avg - mu) ** 2, axis=-1, keepdims=True)
    normed = (avg - mu) * lax.rsqrt(var + 1e-5) * gamma_ref[...] + beta_ref[...]
    h = lax.dot_general(
        normed, w1_ref[...], (((1,), (0,)), ((), ())),
        precision=lax.Precision.HIGHEST,
        preferred_element_type=jnp.float32,
    )
    h = jnp.maximum(h + b1_ref[...], 0.0)
    logits = lax.dot_general(
        h, w2_ref[...], (((1,), (0,)), ((), ())),
        precision=lax.Precision.HIGHEST,
        preferred_element_type=jnp.float32,
    )
    out_ref[...] = logits + b2_ref[...]


def _tail(sums, lengths, row0, gamma, beta, W1, b1, W2, b2):
    BLK = 2048
    return pl.pallas_call(
        _tail_body,
        grid=(B // BLK,),
        in_specs=[
            pl.BlockSpec((BLK, D), lambda i: (i, 0)),
            pl.BlockSpec((BLK, 1), lambda i: (i, 0)),
            pl.BlockSpec((1, D), lambda i: (0, 0)),
            pl.BlockSpec((1, D), lambda i: (0, 0)),
            pl.BlockSpec((1, D), lambda i: (0, 0)),
            pl.BlockSpec((D, HID), lambda i: (0, 0)),
            pl.BlockSpec((1, HID), lambda i: (0, 0)),
            pl.BlockSpec((HID, OUT), lambda i: (0, 0)),
            pl.BlockSpec((1, OUT), lambda i: (0, 0)),
        ],
        out_specs=pl.BlockSpec((BLK, OUT), lambda i: (i, 0)),
        out_shape=jax.ShapeDtypeStruct((B, OUT), jnp.float32),
    )(sums, lengths, row0, gamma, beta, W1, b1, W2, b2)


@jax.jit
def _run(indices, table, gamma, beta, W1, b1, W2, b2):
    table_bf = jnp.pad(table.astype(jnp.bfloat16), ((0, 0), (0, DP - D)))
    sums = _pool(indices, table_bf)
    lengths = _lengths(indices)
    row0 = table[0:1, :].astype(jnp.bfloat16).astype(jnp.float32)
    return _tail(sums, lengths, row0, gamma[None, :], beta[None, :],
                 W1, b1[None, :], W2, b2[None, :])


def kernel(indices, table, gamma, beta, W1, b1, W2, b2):
    return _run(indices, table, gamma, beta, W1, b1, W2, b2)


# speedup vs baseline: 1.6569x; 1.6569x over previous
import jax
import jax.numpy as jnp
from jax import lax
from jax.experimental import pallas as pl
from jax.experimental.pallas import tpu as pltpu
from jax.experimental.pallas import tpu_sc as plsc

B = 16384
L = 200
D = 64
HID = 256
OUT = 2

NC = 2
NS = 16
NW = NC * NS
ROWS_PER_W = B // NW
LANES = 16
W = 64
NG = ROWS_PER_W // W
UNROLL = 8


def _pool_body(idx_hbm, table_hbm, out_hbm, ib, gb0, gb1, ab, sg0, sg1):
    wid = lax.axis_index("s") * NC + lax.axis_index("c")
    base = wid * ROWS_PER_W

    @pl.loop(0, NG)
    def _(g):
        rowbase = base + g * W
        pltpu.sync_copy(idx_hbm.at[pl.ds(rowbase, W)], ib)
        pltpu.async_copy(table_hbm.at[ib.at[0]], gb0, sg0)
        for w in range(W):
            gb, sg = (gb0, sg0) if w % 2 == 0 else (gb1, sg1)
            ngb, nsg = (gb1, sg1) if w % 2 == 0 else (gb0, sg0)
            if w + 1 < W:
                pltpu.async_copy(table_hbm.at[ib.at[w + 1]], ngb, nsg)
            pltpu.make_async_copy(table_hbm.at[ib.at[w]], gb, sg).wait()

            def body(i, acc, gb=gb):
                new = list(acc)
                for u in range(UNROLL):
                    l = i * UNROLL + u
                    for c in range(D // LANES):
                        k = (u % 2) * (D // LANES) + c
                        new[k] = new[k] + gb[l, pl.ds(c * LANES, LANES)]
                return tuple(new)

            zero = jnp.zeros((LANES,), jnp.float32)
            acc = lax.fori_loop(0, L // UNROLL, body,
                                (zero,) * (2 * (D // LANES)))
            for c in range(D // LANES):
                ab[w, pl.ds(c * LANES, LANES)] = acc[c] + acc[D // LANES + c]
        pltpu.sync_copy(ab, out_hbm.at[pl.ds(rowbase, W)])


def _pool(indices, table):
    mesh = plsc.VectorSubcoreMesh(core_axis_name="c", subcore_axis_name="s")
    k = pl.kernel(
        _pool_body,
        out_type=jax.ShapeDtypeStruct((B, D), jnp.float32),
        mesh=mesh,
        compiler_params=pltpu.CompilerParams(
            use_tc_tiling_on_sc=False, needs_layout_passes=False),
        scratch_types=[
            pltpu.VMEM((W, L), jnp.int32),
            pltpu.VMEM((L, D), jnp.float32),
            pltpu.VMEM((L, D), jnp.float32),
            pltpu.VMEM((W, D), jnp.float32),
            pltpu.SemaphoreType.DMA,
            pltpu.SemaphoreType.DMA,
        ],
    )
    return k(indices, table)


def _lengths_body(idx_ref, len_ref):
    idx = idx_ref[...]
    len_ref[...] = jnp.sum((idx != 0).astype(jnp.float32), axis=1,
                           keepdims=True)


def _lengths(indices):
    BLK = 2048
    return pl.pallas_call(
        _lengths_body,
        grid=(B // BLK,),
        in_specs=[pl.BlockSpec((BLK, L), lambda i: (i, 0))],
        out_specs=pl.BlockSpec((BLK, 1), lambda i: (i, 0)),
        out_shape=jax.ShapeDtypeStruct((B, 1), jnp.float32),
    )(indices)


def _tail_body(sums_ref, len_ref, row0_ref, gamma_ref, beta_ref,
               w1_ref, b1_ref, w2_ref, b2_ref, out_ref):
    lengths = len_ref[...]
    n_zeros = jnp.float32(L) - lengths
    s = sums_ref[...] - n_zeros * row0_ref[...]
    avg = s / jnp.maximum(lengths, 1.0)
    mu = jnp.mean(avg, axis=-1, keepdims=True)
    var = jnp.mean((avg - mu) ** 2, axis=-1, keepdims=True)
    normed = (avg - mu) * lax.rsqrt(var + 1e-5) * gamma_ref[...] + beta_ref[...]
    h = lax.dot_general(
        normed, w1_ref[...], (((1,), (0,)), ((), ())),
        preferred_element_type=jnp.float32,
    )
    h = jnp.maximum(h + b1_ref[...], 0.0)
    logits = lax.dot_general(
        h, w2_ref[...], (((1,), (0,)), ((), ())),
        preferred_element_type=jnp.float32,
    )
    out_ref[...] = logits + b2_ref[...]


def _tail(sums, lengths, row0, gamma, beta, W1, b1, W2, b2):
    BLK = 2048
    return pl.pallas_call(
        _tail_body,
        grid=(B // BLK,),
        in_specs=[
            pl.BlockSpec((BLK, D), lambda i: (i, 0)),
            pl.BlockSpec((BLK, 1), lambda i: (i, 0)),
            pl.BlockSpec((1, D), lambda i: (0, 0)),
            pl.BlockSpec((1, D), lambda i: (0, 0)),
            pl.BlockSpec((1, D), lambda i: (0, 0)),
            pl.BlockSpec((D, HID), lambda i: (0, 0)),
            pl.BlockSpec((1, HID), lambda i: (0, 0)),
            pl.BlockSpec((HID, OUT), lambda i: (0, 0)),
            pl.BlockSpec((1, OUT), lambda i: (0, 0)),
        ],
        out_specs=pl.BlockSpec((BLK, OUT), lambda i: (i, 0)),
        out_shape=jax.ShapeDtypeStruct((B, OUT), jnp.float32),
    )(sums, lengths, row0, gamma, beta, W1, b1, W2, b2)


@jax.jit
def _run(indices, table, gamma, beta, W1, b1, W2, b2):
    sums = _pool(indices, table)
    lengths = _lengths(indices)
    row0 = table[0:1, :]
    return _tail(sums, lengths, row0, gamma[None, :], beta[None, :],
                 W1, b1[None, :], W2, b2[None, :])


def kernel(indices, table, gamma, beta, W1, b1, W2, b2):
    return _run(indices, table, gamma, beta, W1, b1, W2, b2)


# speedup vs baseline: 1.7789x; 1.0736x over previous
import jax
import jax.numpy as jnp
from jax import lax
from jax.experimental import pallas as pl
from jax.experimental.pallas import tpu as pltpu
from jax.experimental.pallas import tpu_sc as plsc

B = 16384
L = 200
D = 64
HID = 256
OUT = 2

NC = 2
NS = 16
NW = NC * NS
ROWS_PER_W = B // NW
LANES = 16
W = 64
NG = ROWS_PER_W // W
UNROLL = 8


def _pool_body(idx_hbm, table_hbm, out_hbm, ib, gb0, gb1, ab, sg0, sg1):
    wid = lax.axis_index("s") * NC + lax.axis_index("c")
    base = wid * ROWS_PER_W

    @pl.loop(0, NG)
    def _(g):
        rowbase = base + g * W
        pltpu.sync_copy(idx_hbm.at[pl.ds(rowbase * L, W * L)], ib)
        pltpu.async_copy(table_hbm.at[ib.at[pl.ds(0, 2 * L)]], gb0, sg0)
        for p in range(W // 2):
            gb, sg = (gb0, sg0) if p % 2 == 0 else (gb1, sg1)
            ngb, nsg = (gb1, sg1) if p % 2 == 0 else (gb0, sg0)
            if p + 1 < W // 2:
                pltpu.async_copy(
                    table_hbm.at[ib.at[pl.ds(2 * (p + 1) * L, 2 * L)]], ngb, nsg)
            pltpu.make_async_copy(
                table_hbm.at[ib.at[pl.ds(2 * p * L, 2 * L)]], gb, sg).wait()
            for sub in range(2):
                def body(i, acc, gb=gb, off=sub * L):
                    new = list(acc)
                    for u in range(UNROLL):
                        l = off + i * UNROLL + u
                        for c in range(D // LANES):
                            k = (u % 2) * (D // LANES) + c
                            new[k] = new[k] + gb[l, pl.ds(c * LANES, LANES)]
                    return tuple(new)

                zero = jnp.zeros((LANES,), jnp.float32)
                acc = lax.fori_loop(0, L // UNROLL, body,
                                    (zero,) * (2 * (D // LANES)))
                for c in range(D // LANES):
                    ab[2 * p + sub, pl.ds(c * LANES, LANES)] = (
                        acc[c] + acc[D // LANES + c])
        pltpu.sync_copy(ab, out_hbm.at[pl.ds(rowbase, W)])


def _pool(indices_flat, table):
    mesh = plsc.VectorSubcoreMesh(core_axis_name="c", subcore_axis_name="s")
    k = pl.kernel(
        _pool_body,
        out_type=jax.ShapeDtypeStruct((B, D), jnp.float32),
        mesh=mesh,
        compiler_params=pltpu.CompilerParams(
            use_tc_tiling_on_sc=False, needs_layout_passes=False),
        scratch_types=[
            pltpu.VMEM((W * L,), jnp.int32),
            pltpu.VMEM((2 * L, D), jnp.float32),
            pltpu.VMEM((2 * L, D), jnp.float32),
            pltpu.VMEM((W, D), jnp.float32),
            pltpu.SemaphoreType.DMA,
            pltpu.SemaphoreType.DMA,
        ],
    )
    return k(indices_flat, table)


def _lengths_body(idx_ref, len_ref):
    idx = idx_ref[...]
    len_ref[...] = jnp.sum((idx != 0).astype(jnp.float32), axis=1,
                           keepdims=True)


def _lengths(indices):
    BLK = 2048
    return pl.pallas_call(
        _lengths_body,
        grid=(B // BLK,),
        in_specs=[pl.BlockSpec((BLK, L), lambda i: (i, 0))],
        out_specs=pl.BlockSpec((BLK, 1), lambda i: (i, 0)),
        out_shape=jax.ShapeDtypeStruct((B, 1), jnp.float32),
    )(indices)


def _tail_body(sums_ref, len_ref, row0_ref, gamma_ref, beta_ref,
               w1_ref, b1_ref, w2_ref, b2_ref, out_ref):
    lengths = len_ref[...]
    n_zeros = jnp.float32(L) - lengths
    s = sums_ref[...] - n_zeros * row0_ref[...]
    avg = s / jnp.maximum(lengths, 1.0)
    mu = jnp.mean(avg, axis=-1, keepdims=True)
    var = jnp.mean((avg - mu) ** 2, axis=-1, keepdims=True)
    normed = (avg - mu) * lax.rsqrt(var + 1e-5) * gamma_ref[...] + beta_ref[...]
    h = lax.dot_general(
        normed, w1_ref[...], (((1,), (0,)), ((), ())),
        preferred_element_type=jnp.float32,
    )
    h = jnp.maximum(h + b1_ref[...], 0.0)
    logits = lax.dot_general(
        h, w2_ref[...], (((1,), (0,)), ((), ())),
        preferred_element_type=jnp.float32,
    )
    out_ref[...] = logits + b2_ref[...]


def _tail(sums, lengths, row0, gamma, beta, W1, b1, W2, b2):
    BLK = 2048
    return pl.pallas_call(
        _tail_body,
        grid=(B // BLK,),
        in_specs=[
            pl.BlockSpec((BLK, D), lambda i: (i, 0)),
            pl.BlockSpec((BLK, 1), lambda i: (i, 0)),
            pl.BlockSpec((1, D), lambda i: (0, 0)),
            pl.BlockSpec((1, D), lambda i: (0, 0)),
            pl.BlockSpec((1, D), lambda i: (0, 0)),
            pl.BlockSpec((D, HID), lambda i: (0, 0)),
            pl.BlockSpec((1, HID), lambda i: (0, 0)),
            pl.BlockSpec((HID, OUT), lambda i: (0, 0)),
            pl.BlockSpec((1, OUT), lambda i: (0, 0)),
        ],
        out_specs=pl.BlockSpec((BLK, OUT), lambda i: (i, 0)),
        out_shape=jax.ShapeDtypeStruct((B, OUT), jnp.float32),
    )(sums, lengths, row0, gamma, beta, W1, b1, W2, b2)


@jax.jit
def _run(indices, table, gamma, beta, W1, b1, W2, b2):
    sums = _pool(indices.reshape(-1), table)
    lengths = _lengths(indices)
    row0 = table[0:1, :]
    return _tail(sums, lengths, row0, gamma[None, :], beta[None, :],
                 W1, b1[None, :], W2, b2[None, :])


def kernel(indices, table, gamma, beta, W1, b1, W2, b2):
    return _run(indices, table, gamma, beta, W1, b1, W2, b2)


# speedup vs baseline: 1.8252x; 1.0260x over previous
import jax
import jax.numpy as jnp
from jax import lax
from jax.experimental import pallas as pl
from jax.experimental.pallas import tpu as pltpu
from jax.experimental.pallas import tpu_sc as plsc

B = 16384
L = 200
D = 64
HID = 256
OUT = 2

NC = 2
NS = 16
NW = NC * NS
ROWS_PER_W = B // NW
LANES = 16
W = 64
NG = ROWS_PER_W // W
UNROLL = 8


def _pool_body(idx_hbm, table_hbm, out_hbm, ib, gb0, gb1, ab, sg0, sg1):
    wid = lax.axis_index("s") * NC + lax.axis_index("c")
    base = wid * ROWS_PER_W

    @pl.loop(0, NG)
    def _(g):
        rowbase = base + g * W
        pltpu.sync_copy(idx_hbm.at[pl.ds(rowbase * L, W * L)], ib)
        pltpu.async_copy(table_hbm.at[ib.at[pl.ds(0, 4 * L)]], gb0, sg0)
        for p in range(W // 4):
            gb, sg = (gb0, sg0) if p % 2 == 0 else (gb1, sg1)
            ngb, nsg = (gb1, sg1) if p % 2 == 0 else (gb0, sg0)
            if p + 1 < W // 4:
                pltpu.async_copy(
                    table_hbm.at[ib.at[pl.ds(4 * (p + 1) * L, 4 * L)]], ngb, nsg)
            pltpu.make_async_copy(
                table_hbm.at[ib.at[pl.ds(4 * p * L, 4 * L)]], gb, sg).wait()
            for sub in range(4):
                def body(i, acc, gb=gb, off=sub * L):
                    new = list(acc)
                    for u in range(UNROLL):
                        l = off + i * UNROLL + u
                        for c in range(D // LANES):
                            k = (u % 2) * (D // LANES) + c
                            new[k] = new[k] + gb[l, pl.ds(c * LANES, LANES)]
                    return tuple(new)

                zero = jnp.zeros((LANES,), jnp.float32)
                acc = lax.fori_loop(0, L // UNROLL, body,
                                    (zero,) * (2 * (D // LANES)))
                for c in range(D // LANES):
                    ab[4 * p + sub, pl.ds(c * LANES, LANES)] = (
                        acc[c] + acc[D // LANES + c])
        pltpu.sync_copy(ab, out_hbm.at[pl.ds(rowbase, W)])


def _pool(indices_flat, table):
    mesh = plsc.VectorSubcoreMesh(core_axis_name="c", subcore_axis_name="s")
    k = pl.kernel(
        _pool_body,
        out_type=jax.ShapeDtypeStruct((B, D), jnp.float32),
        mesh=mesh,
        compiler_params=pltpu.CompilerParams(
            use_tc_tiling_on_sc=False, needs_layout_passes=False),
        scratch_types=[
            pltpu.VMEM((W * L,), jnp.int32),
            pltpu.VMEM((4 * L, D), jnp.float32),
            pltpu.VMEM((4 * L, D), jnp.float32),
            pltpu.VMEM((W, D), jnp.float32),
            pltpu.SemaphoreType.DMA,
            pltpu.SemaphoreType.DMA,
        ],
    )
    return k(indices_flat, table)


def _lengths_body(idx_ref, len_ref):
    idx = idx_ref[...]
    len_ref[...] = jnp.sum((idx != 0).astype(jnp.float32), axis=1,
                           keepdims=True)


def _lengths(indices):
    BLK = 2048
    return pl.pallas_call(
        _lengths_body,
        grid=(B // BLK,),
        in_specs=[pl.BlockSpec((BLK, L), lambda i: (i, 0))],
        out_specs=pl.BlockSpec((BLK, 1), lambda i: (i, 0)),
        out_shape=jax.ShapeDtypeStruct((B, 1), jnp.float32),
    )(indices)


def _tail_body(sums_ref, len_ref, row0_ref, gamma_ref, beta_ref,
               w1_ref, b1_ref, w2_ref, b2_ref, out_ref):
    lengths = len_ref[...]
    n_zeros = jnp.float32(L) - lengths
    s = sums_ref[...] - n_zeros * row0_ref[...]
    avg = s / jnp.maximum(lengths, 1.0)
    mu = jnp.mean(avg, axis=-1, keepdims=True)
    var = jnp.mean((avg - mu) ** 2, axis=-1, keepdims=True)
    normed = (avg - mu) * lax.rsqrt(var + 1e-5) * gamma_ref[...] + beta_ref[...]
    h = lax.dot_general(
        normed, w1_ref[...], (((1,), (0,)), ((), ())),
        preferred_element_type=jnp.float32,
    )
    h = jnp.maximum(h + b1_ref[...], 0.0)
    logits = lax.dot_general(
        h, w2_ref[...], (((1,), (0,)), ((), ())),
        preferred_element_type=jnp.float32,
    )
    out_ref[...] = logits + b2_ref[...]


def _tail(sums, lengths, row0, gamma, beta, W1, b1, W2, b2):
    BLK = 2048
    return pl.pallas_call(
        _tail_body,
        grid=(B // BLK,),
        in_specs=[
            pl.BlockSpec((BLK, D), lambda i: (i, 0)),
            pl.BlockSpec((BLK, 1), lambda i: (i, 0)),
            pl.BlockSpec((1, D), lambda i: (0, 0)),
            pl.BlockSpec((1, D), lambda i: (0, 0)),
            pl.BlockSpec((1, D), lambda i: (0, 0)),
            pl.BlockSpec((D, HID), lambda i: (0, 0)),
            pl.BlockSpec((1, HID), lambda i: (0, 0)),
            pl.BlockSpec((HID, OUT), lambda i: (0, 0)),
            pl.BlockSpec((1, OUT), lambda i: (0, 0)),
        ],
        out_specs=pl.BlockSpec((BLK, OUT), lambda i: (i, 0)),
        out_shape=jax.ShapeDtypeStruct((B, OUT), jnp.float32),
    )(sums, lengths, row0, gamma, beta, W1, b1, W2, b2)


@jax.jit
def _run(indices, table, gamma, beta, W1, b1, W2, b2):
    sums = _pool(indices.reshape(-1), table)
    lengths = _lengths(indices)
    row0 = table[0:1, :]
    return _tail(sums, lengths, row0, gamma[None, :], beta[None, :],
                 W1, b1[None, :], W2, b2[None, :])


def kernel(indices, table, gamma, beta, W1, b1, W2, b2):
    return _run(indices, table, gamma, beta, W1, b1, W2, b2)
